# baseline (speedup 1.0000x reference)
import functools

import jax
import jax.numpy as jnp
from jax import lax
from jax.experimental import pallas as pl
from jax.experimental.pallas import tpu as pltpu
from jax.experimental.pallas import tpu_sc as plsc

N = 94
NP = 96
E = 3008
L = 16
EV = E // L
NV = NP // L
UNROLL = 4


def _splat(ref, k):
    return plsc.load_gather(ref, [jnp.full((L,), k, jnp.int32)])


def _rsqrt16(x):
    i = plsc.bitcast(x, jnp.int32)
    i = jnp.int32(0x5F3759DF) - (i >> 1)
    y = plsc.bitcast(i, jnp.float32)
    for _ in range(3):
        y = y * (1.5 - 0.5 * x * y * y)
    return y


def _gcn_body(fh, srch, dsth, ph, outh,
              fv, srcv, dstv, params,
              deg, dinv, xa, xb, ga, gb, xc, gc, normv, hout):
    cid = lax.axis_index("c")
    sid = lax.axis_index("s")

    @pl.when(jnp.logical_and(cid == 0, sid == 0))
    def _():
        pltpu.sync_copy(fh, fv)
        pltpu.sync_copy(srch, srcv)
        pltpu.sync_copy(dsth, dstv)
        pltpu.sync_copy(ph, params)

        zeros = jnp.zeros((L,), jnp.float32)
        ones = jnp.full((L,), 1.0, jnp.float32)

        for i in range(NV):
            deg[pl.ds(i * L, L)] = zeros

        def degbody(i, c):
            for u in range(UNROLL):
                s = pl.ds(pl.multiple_of(i * (L * UNROLL) + u * L, L), L)
                plsc.addupdate_scatter(deg, [dstv[s]], ones)
            return c
        lax.fori_loop(0, EV // UNROLL, degbody, 0)

        for i in range(NV):
            s = pl.ds(i * L, L)
            idx = lax.iota(jnp.int32, L) + i * L
            real = idx < N
            dg = deg[s] + jnp.where(real, 1.0, 0.0)
            dinv[s] = _rsqrt16(jnp.where(real, dg, 1.0))

        w00 = _splat(params, 0)
        w01 = _splat(params, 1)
        w10 = _splat(params, 2)
        w11 = _splat(params, 3)
        w20 = _splat(params, 4)
        w21 = _splat(params, 5)
        for i in range(NV):
            s = pl.ds(i * L, L)
            base = jnp.minimum(lax.iota(jnp.int32, L) + i * L, N - 1) * 3
            f0 = plsc.load_gather(fv, [base])
            f1 = plsc.load_gather(fv, [base + 1])
            f2 = plsc.load_gather(fv, [base + 2])
            xa[s] = f0 * w00 + f1 * w10 + f2 * w20
            xb[s] = f0 * w01 + f1 * w11 + f2 * w21
            ga[s] = zeros
            gb[s] = zeros

        def e1(i, c):
            for u in range(UNROLL):
                s = pl.ds(pl.multiple_of(i * (L * UNROLL) + u * L, L), L)
                sv = srcv[s]
                dv = dstv[s]
                nm = plsc.load_gather(dinv, [sv]) * plsc.load_gather(dinv, [dv])
                normv[s] = nm
                plsc.addupdate_scatter(ga, [dv], nm * plsc.load_gather(xa, [sv]))
                plsc.addupdate_scatter(gb, [dv], nm * plsc.load_gather(xb, [sv]))
            return c
        lax.fori_loop(0, EV // UNROLL, e1, 0)

        b1a = _splat(params, 6)
        b1b = _splat(params, 7)
        w2a = _splat(params, 8)
        w2b = _splat(params, 9)
        for i in range(NV):
            s = pl.ds(i * L, L)
            dv2 = dinv[s] * dinv[s]
            va = jnp.maximum(ga[s] + dv2 * xa[s] + b1a, 0.0)
            vb = jnp.maximum(gb[s] + dv2 * xb[s] + b1b, 0.0)
            xc[s] = va * w2a + vb * w2b
            gc[s] = zeros

        def e2(i, c):
            for u in range(UNROLL):
                s = pl.ds(pl.multiple_of(i * (L * UNROLL) + u * L, L), L)
                sv = srcv[s]
                dv = dstv[s]
                plsc.addupdate_scatter(gc, [dv],
                                       normv[s] * plsc.load_gather(xc, [sv]))
            return c
        lax.fori_loop(0, EV // UNROLL, e2, 0)

        b2s = _splat(params, 10)
        for i in range(NV):
            s = pl.ds(i * L, L)
            idx = lax.iota(jnp.int32, L) + i * L
            dv2 = dinv[s] * dinv[s]
            hv = jnp.maximum(gc[s] + dv2 * xc[s] + b2s, 0.0)
            hout[s] = jnp.where(idx < N, hv, 0.0)

        pltpu.sync_copy(hout, outh)


_SC_SCRATCH = [
    pltpu.VMEM((288,), jnp.float32),
    pltpu.VMEM((E,), jnp.int32),
    pltpu.VMEM((E,), jnp.int32),
    pltpu.VMEM((L,), jnp.float32),
    pltpu.VMEM((NP,), jnp.float32),
    pltpu.VMEM((NP,), jnp.float32),
    pltpu.VMEM((NP,), jnp.float32),
    pltpu.VMEM((NP,), jnp.float32),
    pltpu.VMEM((NP,), jnp.float32),
    pltpu.VMEM((NP,), jnp.float32),
    pltpu.VMEM((NP,), jnp.float32),
    pltpu.VMEM((NP,), jnp.float32),
    pltpu.VMEM((E,), jnp.float32),
    pltpu.VMEM((NP,), jnp.float32),
]


def _sc_gcn(feature, src, dst, params):
    fn = functools.partial(
        pl.kernel,
        out_type=jax.ShapeDtypeStruct((NP,), jnp.float32),
        mesh=plsc.VectorSubcoreMesh(core_axis_name="c", subcore_axis_name="s"),
        scratch_types=_SC_SCRATCH,
        compiler_params=pltpu.CompilerParams(needs_layout_passes=False),
    )(_gcn_body)
    return fn(feature, src, dst, params)


BLK = 640
NBLK = 6400 // BLK


def _tc_body(h_ref, w1_ref, b1_ref, w2_ref, b2_ref, w3_ref, b3_ref,
             o_ref, h2_ref):
    @pl.when(pl.program_id(0) == 0)
    def _():
        x = h_ref[...]
        w1 = jnp.concatenate(
            [w1_ref[...], jnp.zeros((NP - N, 512), jnp.float32)])
        h1 = jnp.sum(x * w1, axis=0, keepdims=True)
        h1 = jnp.maximum(h1 + b1_ref[...], 0.0)
        h2 = lax.dot_general(w2_ref[...], h1,
                             (((0,), (1,)), ((), ())),
                             preferred_element_type=jnp.float32)
        h2_ref[...] = jnp.maximum(h2 + b2_ref[...], 0.0)

    acc = h2_ref[...] * w3_ref[...]
    for sz in (512, 256, 128, 64, 32, 16, 8):
        acc = acc[:sz] + acc[sz:]
    o_ref[...] = jnp.sum(acc, axis=0, keepdims=True) + b3_ref[...]


def _tc_mlp(hcol, w1, b1r, w2, b2c, w3, b3r):
    return pl.pallas_call(
        _tc_body,
        grid=(NBLK,),
        in_specs=[
            pl.BlockSpec((NP, 1), lambda j: (0, 0)),
            pl.BlockSpec((N, 512), lambda j: (0, 0)),
            pl.BlockSpec((1, 512), lambda j: (0, 0)),
            pl.BlockSpec((512, 1024), lambda j: (0, 0)),
            pl.BlockSpec((1024, 1), lambda j: (0, 0)),
            pl.BlockSpec((1024, BLK), lambda j: (0, j)),
            pl.BlockSpec((1, BLK), lambda j: (0, j)),
        ],
        out_specs=pl.BlockSpec((1, BLK), lambda j: (0, j)),
        out_shape=jax.ShapeDtypeStruct((1, 6400), jnp.float32),
        scratch_shapes=[pltpu.VMEM((1024, 1), jnp.float32)],
        compiler_params=pltpu.CompilerParams(
            dimension_semantics=("arbitrary",)),
    )(hcol, w1, b1r, w2, b2c, w3, b3r)


def kernel(feature, edge_index, W1, b1, W2, b2, Wfc1, bfc1, Wfc2, bfc2, Wfc, bfc):
    params = jnp.concatenate(
        [W1.reshape(-1), b1, W2.reshape(-1), b2,
         jnp.zeros((L - 11,), jnp.float32)])
    h96 = _sc_gcn(jnp.pad(feature.reshape(-1), (0, 288 - 3 * N)),
                  edge_index[0], edge_index[1], params)
    out = _tc_mlp(h96.reshape(NP, 1),
                  Wfc1,
                  bfc1.reshape(1, -1),
                  Wfc2,
                  bfc2.reshape(-1, 1),
                  Wfc,
                  bfc.reshape(1, -1))
    return out.reshape(-1)


# speedup vs baseline: 1.0273x; 1.0273x over previous
import functools

import jax
import jax.numpy as jnp
from jax import lax
from jax.experimental import pallas as pl
from jax.experimental.pallas import tpu as pltpu
from jax.experimental.pallas import tpu_sc as plsc

N = 94
NP = 96
E = 3008
L = 16
EV = E // L
NV = NP // L
UNROLL = 4


def _splat(ref, k):
    return plsc.load_gather(ref, [jnp.full((L,), k, jnp.int32)])


def _rsqrt16(x):
    i = plsc.bitcast(x, jnp.int32)
    i = jnp.int32(0x5F3759DF) - (i >> 1)
    y = plsc.bitcast(i, jnp.float32)
    for _ in range(3):
        y = y * (1.5 - 0.5 * x * y * y)
    return y


def _gcn_body(fh, srch, dsth, ph, outh,
              fv, srcv, dstv, params,
              deg, dinv, xa, xb, ga, gb, xc, gc, normv, hout):
    cid = lax.axis_index("c")
    sid = lax.axis_index("s")

    @pl.when(jnp.logical_and(cid == 0, sid == 0))
    def _():
        pltpu.sync_copy(fh, fv)
        pltpu.sync_copy(srch, srcv)
        pltpu.sync_copy(dsth, dstv)
        pltpu.sync_copy(ph, params)

        zeros = jnp.zeros((L,), jnp.float32)
        ones = jnp.full((L,), 1.0, jnp.float32)

        for i in range(NV):
            deg[pl.ds(i * L, L)] = zeros

        def degbody(i, c):
            for u in range(UNROLL):
                s = pl.ds(pl.multiple_of(i * (L * UNROLL) + u * L, L), L)
                plsc.addupdate_scatter(deg, [dstv[s]], ones)
            return c
        lax.fori_loop(0, EV // UNROLL, degbody, 0)

        for i in range(NV):
            s = pl.ds(i * L, L)
            idx = lax.iota(jnp.int32, L) + i * L
            real = idx < N
            dg = deg[s] + jnp.where(real, 1.0, 0.0)
            dinv[s] = _rsqrt16(jnp.where(real, dg, 1.0))

        w00 = _splat(params, 0)
        w01 = _splat(params, 1)
        w10 = _splat(params, 2)
        w11 = _splat(params, 3)
        w20 = _splat(params, 4)
        w21 = _splat(params, 5)
        for i in range(NV):
            s = pl.ds(i * L, L)
            base = jnp.minimum(lax.iota(jnp.int32, L) + i * L, N - 1) * 3
            f0 = plsc.load_gather(fv, [base])
            f1 = plsc.load_gather(fv, [base + 1])
            f2 = plsc.load_gather(fv, [base + 2])
            xa[s] = f0 * w00 + f1 * w10 + f2 * w20
            xb[s] = f0 * w01 + f1 * w11 + f2 * w21
            ga[s] = zeros
            gb[s] = zeros

        def e1(i, c):
            for u in range(UNROLL):
                s = pl.ds(pl.multiple_of(i * (L * UNROLL) + u * L, L), L)
                sv = srcv[s]
                dv = dstv[s]
                nm = plsc.load_gather(dinv, [sv]) * plsc.load_gather(dinv, [dv])
                normv[s] = nm
                plsc.addupdate_scatter(ga, [dv], nm * plsc.load_gather(xa, [sv]))
                plsc.addupdate_scatter(gb, [dv], nm * plsc.load_gather(xb, [sv]))
            return c
        lax.fori_loop(0, EV // UNROLL, e1, 0)

        b1a = _splat(params, 6)
        b1b = _splat(params, 7)
        w2a = _splat(params, 8)
        w2b = _splat(params, 9)
        for i in range(NV):
            s = pl.ds(i * L, L)
            dv2 = dinv[s] * dinv[s]
            va = jnp.maximum(ga[s] + dv2 * xa[s] + b1a, 0.0)
            vb = jnp.maximum(gb[s] + dv2 * xb[s] + b1b, 0.0)
            xc[s] = va * w2a + vb * w2b
            gc[s] = zeros

        def e2(i, c):
            for u in range(UNROLL):
                s = pl.ds(pl.multiple_of(i * (L * UNROLL) + u * L, L), L)
                sv = srcv[s]
                dv = dstv[s]
                plsc.addupdate_scatter(gc, [dv],
                                       normv[s] * plsc.load_gather(xc, [sv]))
            return c
        lax.fori_loop(0, EV // UNROLL, e2, 0)

        b2s = _splat(params, 10)
        for i in range(NV):
            s = pl.ds(i * L, L)
            idx = lax.iota(jnp.int32, L) + i * L
            dv2 = dinv[s] * dinv[s]
            hv = jnp.maximum(gc[s] + dv2 * xc[s] + b2s, 0.0)
            hout[s] = jnp.where(idx < N, hv, 0.0)

        pltpu.sync_copy(hout, outh)


_SC_SCRATCH = [
    pltpu.VMEM((288,), jnp.float32),
    pltpu.VMEM((E,), jnp.int32),
    pltpu.VMEM((E,), jnp.int32),
    pltpu.VMEM((L,), jnp.float32),
    pltpu.VMEM((NP,), jnp.float32),
    pltpu.VMEM((NP,), jnp.float32),
    pltpu.VMEM((NP,), jnp.float32),
    pltpu.VMEM((NP,), jnp.float32),
    pltpu.VMEM((NP,), jnp.float32),
    pltpu.VMEM((NP,), jnp.float32),
    pltpu.VMEM((NP,), jnp.float32),
    pltpu.VMEM((NP,), jnp.float32),
    pltpu.VMEM((E,), jnp.float32),
    pltpu.VMEM((NP,), jnp.float32),
]


def _sc_gcn(feature, src, dst, params):
    fn = functools.partial(
        pl.kernel,
        out_type=jax.ShapeDtypeStruct((NP,), jnp.float32),
        mesh=plsc.VectorSubcoreMesh(core_axis_name="c", subcore_axis_name="s"),
        scratch_types=_SC_SCRATCH,
        compiler_params=pltpu.CompilerParams(needs_layout_passes=False),
    )(_gcn_body)
    return fn(feature, src, dst, params)


KBLK = 128
NBLK = 1024 // KBLK


def _tc_body(h_ref, w1_ref, b1_ref, w2_ref, b2_ref, w3_ref, b3_ref,
             o_ref, h2_ref):
    i = pl.program_id(0)

    @pl.when(i == 0)
    def _():
        x = h_ref[...]
        w1 = jnp.concatenate(
            [w1_ref[...], jnp.zeros((NP - N, 512), jnp.float32)])
        h1 = jnp.sum(x * w1, axis=0, keepdims=True)
        h1 = jnp.maximum(h1 + b1_ref[...], 0.0)
        h2 = lax.dot_general(w2_ref[...], h1,
                             (((0,), (1,)), ((), ())),
                             preferred_element_type=jnp.float32)
        h2_ref[...] = jnp.maximum(h2 + b2_ref[...], 0.0)

    acc = h2_ref[pl.ds(i * KBLK, KBLK), :] * w3_ref[...]
    for sz in (64, 32, 16, 8):
        acc = acc[:sz] + acc[sz:]
    part = jnp.sum(acc, axis=0, keepdims=True)

    @pl.when(i == 0)
    def _():
        o_ref[...] = part + b3_ref[...]

    @pl.when(i > 0)
    def _():
        o_ref[...] += part


def _tc_mlp(hcol, w1, b1r, w2, b2c, w3, b3r):
    return pl.pallas_call(
        _tc_body,
        grid=(NBLK,),
        in_specs=[
            pl.BlockSpec((NP, 1), lambda j: (0, 0)),
            pl.BlockSpec((N, 512), lambda j: (0, 0)),
            pl.BlockSpec((1, 512), lambda j: (0, 0)),
            pl.BlockSpec((512, 1024), lambda j: (0, 0)),
            pl.BlockSpec((1024, 1), lambda j: (0, 0)),
            pl.BlockSpec((KBLK, 6400), lambda j: (j, 0)),
            pl.BlockSpec((1, 6400), lambda j: (0, 0)),
        ],
        out_specs=pl.BlockSpec((1, 6400), lambda j: (0, 0)),
        out_shape=jax.ShapeDtypeStruct((1, 6400), jnp.float32),
        scratch_shapes=[pltpu.VMEM((1024, 1), jnp.float32)],
        compiler_params=pltpu.CompilerParams(
            dimension_semantics=("arbitrary",)),
    )(hcol, w1, b1r, w2, b2c, w3, b3r)


def kernel(feature, edge_index, W1, b1, W2, b2, Wfc1, bfc1, Wfc2, bfc2, Wfc, bfc):
    params = jnp.concatenate(
        [W1.reshape(-1), b1, W2.reshape(-1), b2,
         jnp.zeros((L - 11,), jnp.float32)])
    h96 = _sc_gcn(jnp.pad(feature.reshape(-1), (0, 288 - 3 * N)),
                  edge_index[0], edge_index[1], params)
    out = _tc_mlp(h96.reshape(NP, 1),
                  Wfc1,
                  bfc1.reshape(1, -1),
                  Wfc2,
                  bfc2.reshape(-1, 1),
                  Wfc,
                  bfc.reshape(1, -1))
    return out.reshape(-1)
